# Initial kernel scaffold; baseline (speedup 1.0000x reference)
import jax
import jax.numpy as jnp
from jax.experimental import pallas as pl


def kernel(x, durations, target_len):
    raise NotImplementedError("write your pallas kernel here")


# speedup vs baseline: 89.9997x; 89.9997x over previous
import functools

import jax
import jax.numpy as jnp
from jax import lax
from jax.experimental import pallas as pl
from jax.experimental.pallas import tpu as pltpu
from jax.experimental.pallas import tpu_sc as plsc

_L = 16
_T_OUT = 4096
_CHUNK = 128


def _lr_body(B, N, D, T, x_hbm, dur_hbm, out_hbm, mask_hbm,
             dur_v, A_v, fidx_v, mask_v, gbuf, zbuf, gsem):
  half = T // 2
  nchunk = half // _CHUNK
  wid = lax.axis_index("s") * 2 + lax.axis_index("c")
  b = wid // 2
  h = wid % 2

  pltpu.sync_copy(dur_hbm.at[b], dur_v)

  zeros_i = jnp.zeros((_L,), jnp.int32)
  zeros_f = jnp.zeros((_L,), jnp.float32)
  iota = lax.iota(jnp.int32, _L)

  def zero_a(i, _):
    A_v[pl.ds(i * _L, _L)] = zeros_i
    return 0
  lax.fori_loop(0, T // _L, zero_a, 0)

  def zero_z(i, _):
    for v in range(D // _L):
      zbuf[i, pl.ds(v * _L, _L)] = zeros_f
    return 0
  lax.fori_loop(0, _CHUNK, zero_z, 0)

  def scan_dur(i, carry):
    v = dur_v[pl.ds(i * _L, _L)]
    s = plsc.cumsum(v) + carry
    start = s - v
    m = (v > 0) & (start < T)
    plsc.store_scatter(A_v, [jnp.minimum(start, T - 1)], i * _L + iota, mask=m)
    return jnp.max(s)
  total = lax.fori_loop(0, N // _L, scan_dur, jnp.int32(0))

  base = b * N

  def scan_frames(j, carry):
    a = A_v[pl.ds(j * _L, _L)]
    idxv = jnp.maximum(plsc.cummax(a), carry)
    tvec = j * _L + iota
    fidx_v[pl.ds(j * _L, _L)] = base + idxv
    mask_v[pl.ds(j * _L, _L)] = (tvec < total).astype(jnp.int32)
    return jnp.max(idxv)
  lax.fori_loop(0, T // _L, scan_frames, jnp.int32(0))

  pltpu.sync_copy(mask_v.at[pl.ds(h * half, half)],
                  mask_hbm.at[pl.ds(b * T + h * half, half)])

  def do_chunk(g, _):
    fs = h * half + g * _CHUNK
    gbase = b * T + fs
    nvalid = jnp.clip(total - fs, 0, _CHUNK)

    @pl.when(nvalid > 0)
    def _():
      pltpu.async_copy(x_hbm.at[fidx_v.at[pl.ds(fs, _CHUNK)]], gbuf, gsem).wait()

      def zrow(r, _):
        for v in range(D // _L):
          gbuf[r, pl.ds(v * _L, _L)] = zeros_f
        return 0
      lax.fori_loop(nvalid, _CHUNK, zrow, 0)
      pltpu.sync_copy(gbuf, out_hbm.at[pl.ds(gbase, _CHUNK)])

    @pl.when(nvalid <= 0)
    def _():
      pltpu.sync_copy(zbuf, out_hbm.at[pl.ds(gbase, _CHUNK)])
    return 0
  lax.fori_loop(0, nchunk, do_chunk, 0)


def kernel(x, durations, target_len):
  B, N, D = x.shape
  T = _T_OUT
  bound = jnp.minimum(jnp.asarray(target_len, jnp.int32), T)
  cum = jnp.cumsum(durations.astype(jnp.int32), axis=1)
  cumc = jnp.minimum(cum, bound)
  durc = jnp.concatenate([cumc[:, :1], cumc[:, 1:] - cumc[:, :-1]], axis=1)

  mesh = plsc.VectorSubcoreMesh(core_axis_name="c", subcore_axis_name="s")
  out_flat, mask_flat = pl.kernel(
      functools.partial(_lr_body, B, N, D, T),
      out_type=(jax.ShapeDtypeStruct((B * T, D), jnp.float32),
                jax.ShapeDtypeStruct((B * T,), jnp.int32)),
      mesh=mesh,
      compiler_params=pltpu.CompilerParams(needs_layout_passes=False),
      scratch_types=[
          pltpu.VMEM((N,), jnp.int32),
          pltpu.VMEM((T,), jnp.int32),
          pltpu.VMEM((T,), jnp.int32),
          pltpu.VMEM((T,), jnp.int32),
          pltpu.VMEM((_CHUNK, D), jnp.float32),
          pltpu.VMEM((_CHUNK, D), jnp.float32),
          pltpu.SemaphoreType.DMA,
      ],
  )(x.reshape(B * N, D), durc)
  return out_flat.reshape(B, T, D), (mask_flat.reshape(B, T) != 0)


# speedup vs baseline: 111.1583x; 1.2351x over previous
import functools

import jax
import jax.numpy as jnp
from jax import lax
from jax.experimental import pallas as pl
from jax.experimental.pallas import tpu as pltpu
from jax.experimental.pallas import tpu_sc as plsc

_L = 16
_T_OUT = 4096
_CHUNK = 128


def _lr_body(B, N, D, T, x_hbm, dur_hbm, out_hbm, mask_hbm,
             dur_v, A_v, fidx_v, mask_v, gbuf, zbuf, gsem, osem, zsem):
  half = T // 2
  nchunk = half // _CHUNK
  wid = lax.axis_index("s") * 2 + lax.axis_index("c")
  b = wid // 2
  h = wid % 2

  pltpu.sync_copy(dur_hbm.at[b], dur_v)

  zeros_i = jnp.zeros((_L,), jnp.int32)
  zeros_f = jnp.zeros((_L,), jnp.float32)
  iota = lax.iota(jnp.int32, _L)

  def zero_a(i, _):
    A_v[pl.ds(i * _L, _L)] = zeros_i
    return 0
  lax.fori_loop(0, T // _L, zero_a, 0)

  def zero_z(i, _):
    for v in range(D // _L):
      zbuf[i, pl.ds(v * _L, _L)] = zeros_f
    return 0
  lax.fori_loop(0, _CHUNK, zero_z, 0)

  def scan_dur(i, carry):
    v = dur_v[pl.ds(i * _L, _L)]
    s = plsc.cumsum(v) + carry
    start = s - v
    m = (v > 0) & (start < T)
    plsc.store_scatter(A_v, [jnp.minimum(start, T - 1)], i * _L + iota, mask=m)
    return jnp.max(s)
  total = lax.fori_loop(0, N // _L, scan_dur, jnp.int32(0))

  dvalid = total - h * half
  gv = jnp.clip((dvalid + _CHUNK - 1) // _CHUNK, 0, nchunk)

  def obase(g):
    return b * T + h * half + g * _CHUNK

  def zfill(g, _):
    pltpu.async_copy(zbuf, out_hbm.at[pl.ds(obase(g), _CHUNK)], zsem)
    return 0
  lax.fori_loop(gv, nchunk, zfill, 0)

  base = b * N

  def scan_frames(j, carry):
    a = A_v[pl.ds(j * _L, _L)]
    idxv = jnp.maximum(plsc.cummax(a), carry)
    tvec = j * _L + iota
    fidx_v[pl.ds(j * _L, _L)] = base + idxv
    mask_v[pl.ds(j * _L, _L)] = (tvec < total).astype(jnp.int32)
    return jnp.max(idxv)
  lax.fori_loop(0, (h + 1) * (half // _L), scan_frames, jnp.int32(0))

  def gstart(g, p):
    pltpu.async_copy(
        x_hbm.at[fidx_v.at[pl.ds(h * half + g * _CHUNK, _CHUNK)]],
        gbuf.at[p], gsem.at[p])

  def gwait(g, p):
    pltpu.make_async_copy(
        x_hbm.at[fidx_v.at[pl.ds(h * half + g * _CHUNK, _CHUNK)]],
        gbuf.at[p], gsem.at[p]).wait()

  def ostart(g, p):
    pltpu.async_copy(gbuf.at[p], out_hbm.at[pl.ds(obase(g), _CHUNK)],
                     osem.at[p])

  def owait(g, p):
    pltpu.make_async_copy(gbuf.at[p], out_hbm.at[pl.ds(obase(g), _CHUNK)],
                          osem.at[p]).wait()

  @pl.when(gv > 0)
  def _():
    gstart(0, 0)

  def pipe(g, _):
    p = g % 2
    q = 1 - p

    @pl.when(g + 1 < gv)
    def _():
      @pl.when(g >= 1)
      def _():
        owait(g - 1, q)
      gstart(g + 1, q)

    gwait(g, p)
    nvalid = jnp.clip(total - (h * half + g * _CHUNK), 0, _CHUNK)

    def zrow(r, _):
      for v in range(D // _L):
        gbuf[p, r, pl.ds(v * _L, _L)] = zeros_f
      return 0
    lax.fori_loop(nvalid, _CHUNK, zrow, 0)
    ostart(g, p)
    return 0
  lax.fori_loop(0, gv, pipe, 0)

  @pl.when(gv >= 2)
  def _():
    owait(gv - 2, gv % 2)

  @pl.when(gv >= 1)
  def _():
    owait(gv - 1, (gv + 1) % 2)

  def zdrain(i, _):
    pltpu.make_async_copy(zbuf, out_hbm.at[pl.ds(obase(gv + i), _CHUNK)],
                          zsem).wait()
    return 0
  lax.fori_loop(0, nchunk - gv, zdrain, 0)

  pltpu.sync_copy(mask_v.at[pl.ds(h * half, half)],
                  mask_hbm.at[pl.ds(b * T + h * half, half)])


def kernel(x, durations, target_len):
  B, N, D = x.shape
  T = _T_OUT
  bound = jnp.minimum(jnp.asarray(target_len, jnp.int32), T)
  cum = jnp.cumsum(durations.astype(jnp.int32), axis=1)
  cumc = jnp.minimum(cum, bound)
  durc = jnp.concatenate([cumc[:, :1], cumc[:, 1:] - cumc[:, :-1]], axis=1)

  mesh = plsc.VectorSubcoreMesh(core_axis_name="c", subcore_axis_name="s")
  out_flat, mask_flat = pl.kernel(
      functools.partial(_lr_body, B, N, D, T),
      out_type=(jax.ShapeDtypeStruct((B * T, D), jnp.float32),
                jax.ShapeDtypeStruct((B * T,), jnp.int32)),
      mesh=mesh,
      compiler_params=pltpu.CompilerParams(needs_layout_passes=False),
      scratch_types=[
          pltpu.VMEM((N,), jnp.int32),
          pltpu.VMEM((T,), jnp.int32),
          pltpu.VMEM((T,), jnp.int32),
          pltpu.VMEM((T,), jnp.int32),
          pltpu.VMEM((2, _CHUNK, D), jnp.float32),
          pltpu.VMEM((_CHUNK, D), jnp.float32),
          pltpu.SemaphoreType.DMA((2,)),
          pltpu.SemaphoreType.DMA((2,)),
          pltpu.SemaphoreType.DMA,
      ],
  )(x.reshape(B * N, D), durc)
  return out_flat.reshape(B, T, D), (mask_flat.reshape(B, T) != 0)


# speedup vs baseline: 113.0029x; 1.0166x over previous
import functools

import jax
import jax.numpy as jnp
from jax import lax
from jax.experimental import pallas as pl
from jax.experimental.pallas import tpu as pltpu
from jax.experimental.pallas import tpu_sc as plsc

_L = 16
_T_OUT = 4096
_CHUNK = 128


def _lr_body(B, N, D, T, x_hbm, dur_hbm, out_hbm, mask_hbm,
             dur_v, A_v, fidx_v, mask_v, gbuf, zbuf, gsem, osem, zsem):
  half = T // 2
  nchunk = half // _CHUNK
  wid = lax.axis_index("s") * 2 + lax.axis_index("c")
  b = wid // 2
  h = wid % 2

  pltpu.sync_copy(dur_hbm.at[b], dur_v)

  zeros_i = jnp.zeros((_L,), jnp.int32)
  zeros_f = jnp.zeros((_L,), jnp.float32)
  iota = lax.iota(jnp.int32, _L)

  def zero_a(i, _):
    A_v[pl.ds(i * _L, _L)] = zeros_i
    return 0
  lax.fori_loop(0, T // _L, zero_a, 0)

  def zero_z(i, _):
    for v in range(D // _L):
      zbuf[i, pl.ds(v * _L, _L)] = zeros_f
    return 0
  lax.fori_loop(0, _CHUNK, zero_z, 0)

  def scan_dur(i, carry):
    v = dur_v[pl.ds(i * _L, _L)]
    s = plsc.cumsum(v) + carry
    start = s - v
    m = (v > 0) & (start < T)
    plsc.store_scatter(A_v, [jnp.minimum(start, T - 1)], i * _L + iota, mask=m)
    return jnp.max(s)
  total = lax.fori_loop(0, N // _L, scan_dur, jnp.int32(0))

  vc = (total + _CHUNK - 1) // _CHUNK
  gv = jnp.clip((vc + 1 - h) // 2, 0, nchunk)

  def cbase(g):
    return (h + 2 * g) * _CHUNK

  def zfill(g, _):
    pltpu.async_copy(zbuf, out_hbm.at[pl.ds(b * T + cbase(g), _CHUNK)], zsem)
    return 0
  lax.fori_loop(gv, nchunk, zfill, 0)

  def mrow(j, _):
    tvec = h * half + j * _L + iota
    mask_v[pl.ds(j * _L, _L)] = (tvec < total).astype(jnp.int32)
    return 0
  lax.fori_loop(0, half // _L, mrow, 0)
  pltpu.sync_copy(mask_v, mask_hbm.at[pl.ds(b * T + h * half, half)])

  base = b * N
  sgroups = jnp.minimum((vc + 1) * (_CHUNK // _L), T // _L)

  def scan_frames(j, carry):
    a = A_v[pl.ds(j * _L, _L)]
    idxv = jnp.maximum(plsc.cummax(a), carry)
    fidx_v[pl.ds(j * _L, _L)] = base + idxv
    return jnp.max(idxv)
  lax.fori_loop(0, sgroups, scan_frames, jnp.int32(0))

  def gstart(g, p):
    pltpu.async_copy(x_hbm.at[fidx_v.at[pl.ds(cbase(g), _CHUNK)]],
                     gbuf.at[p], gsem.at[p])

  def gwait(g, p):
    pltpu.make_async_copy(x_hbm.at[fidx_v.at[pl.ds(cbase(g), _CHUNK)]],
                          gbuf.at[p], gsem.at[p]).wait()

  def ostart(g, p):
    pltpu.async_copy(gbuf.at[p], out_hbm.at[pl.ds(b * T + cbase(g), _CHUNK)],
                     osem.at[p])

  def owait(g, p):
    pltpu.make_async_copy(gbuf.at[p],
                          out_hbm.at[pl.ds(b * T + cbase(g), _CHUNK)],
                          osem.at[p]).wait()

  @pl.when(gv > 0)
  def _():
    gstart(0, 0)

  def pipe(g, _):
    p = g % 2
    q = 1 - p

    @pl.when(g + 1 < gv)
    def _():
      @pl.when(g >= 1)
      def _():
        owait(g - 1, q)
      gstart(g + 1, q)

    gwait(g, p)
    nvalid = jnp.clip(total - cbase(g), 0, _CHUNK)

    def zrow(r, _):
      for v in range(D // _L):
        gbuf[p, r, pl.ds(v * _L, _L)] = zeros_f
      return 0
    lax.fori_loop(nvalid, _CHUNK, zrow, 0)
    ostart(g, p)
    return 0
  lax.fori_loop(0, gv, pipe, 0)

  @pl.when(gv >= 2)
  def _():
    owait(gv - 2, gv % 2)

  @pl.when(gv >= 1)
  def _():
    owait(gv - 1, (gv + 1) % 2)

  def zdrain(i, _):
    pltpu.make_async_copy(zbuf, out_hbm.at[pl.ds(b * T + cbase(gv + i), _CHUNK)],
                          zsem).wait()
    return 0
  lax.fori_loop(0, nchunk - gv, zdrain, 0)


def kernel(x, durations, target_len):
  B, N, D = x.shape
  T = _T_OUT
  bound = jnp.minimum(jnp.asarray(target_len, jnp.int32), T)
  cum = jnp.cumsum(durations.astype(jnp.int32), axis=1)
  cumc = jnp.minimum(cum, bound)
  durc = jnp.concatenate([cumc[:, :1], cumc[:, 1:] - cumc[:, :-1]], axis=1)

  mesh = plsc.VectorSubcoreMesh(core_axis_name="c", subcore_axis_name="s")
  out_flat, mask_flat = pl.kernel(
      functools.partial(_lr_body, B, N, D, T),
      out_type=(jax.ShapeDtypeStruct((B * T, D), jnp.float32),
                jax.ShapeDtypeStruct((B * T,), jnp.int32)),
      mesh=mesh,
      compiler_params=pltpu.CompilerParams(needs_layout_passes=False),
      scratch_types=[
          pltpu.VMEM((N,), jnp.int32),
          pltpu.VMEM((T,), jnp.int32),
          pltpu.VMEM((T,), jnp.int32),
          pltpu.VMEM((T // 2,), jnp.int32),
          pltpu.VMEM((2, _CHUNK, D), jnp.float32),
          pltpu.VMEM((_CHUNK, D), jnp.float32),
          pltpu.SemaphoreType.DMA((2,)),
          pltpu.SemaphoreType.DMA((2,)),
          pltpu.SemaphoreType.DMA,
      ],
  )(x.reshape(B * N, D), durc)
  return out_flat.reshape(B, T, D), (mask_flat.reshape(B, T) != 0)


# speedup vs baseline: 117.1879x; 1.0370x over previous
import functools

import jax
import jax.numpy as jnp
from jax import lax
from jax.experimental import pallas as pl
from jax.experimental.pallas import tpu as pltpu
from jax.experimental.pallas import tpu_sc as plsc

_L = 16
_T_OUT = 4096
_CHUNK = 128


def _lr_body(B, N, D, T, x_hbm, dur_hbm, tl_hbm, out_hbm, mask_hbm,
             dur_v, tl_v, A_v, fidx_v, mask_v, gbuf, zbuf, gsem, osem, zsem):
  half = T // 2
  nchunk = half // _CHUNK
  wid = lax.axis_index("s") * 2 + lax.axis_index("c")
  b = wid // 2
  h = wid % 2

  pltpu.sync_copy(dur_hbm.at[b], dur_v)
  pltpu.sync_copy(tl_hbm, tl_v)
  tl_s = jnp.max(tl_v[...])

  zeros_i = jnp.zeros((_L,), jnp.int32)
  zeros_f = jnp.zeros((_L,), jnp.float32)
  iota = lax.iota(jnp.int32, _L)

  def zero_a(i, _):
    A_v[pl.ds(i * _L, _L)] = zeros_i
    return 0
  lax.fori_loop(0, T // _L, zero_a, 0)

  def zero_z(i, _):
    for v in range(D // _L):
      zbuf[i, pl.ds(v * _L, _L)] = zeros_f
    return 0
  lax.fori_loop(0, _CHUNK, zero_z, 0)

  def scan_dur(i, carry):
    v = dur_v[pl.ds(i * _L, _L)]
    s = plsc.cumsum(v) + carry
    s_c = jnp.minimum(s, tl_s)
    start = jnp.minimum(s - v, tl_s)
    m = (s_c > start) & (start < T)
    plsc.store_scatter(A_v, [jnp.minimum(start, T - 1)], i * _L + iota, mask=m)
    return jnp.max(s)
  raw_total = lax.fori_loop(0, N // _L, scan_dur, jnp.int32(0))
  total = jnp.minimum(raw_total, tl_s)

  vc = (total + _CHUNK - 1) // _CHUNK
  gv = jnp.clip((vc + 1 - h) // 2, 0, nchunk)

  def cbase(g):
    return (h + 2 * g) * _CHUNK

  def zfill(g, _):
    pltpu.async_copy(zbuf, out_hbm.at[pl.ds(b * T + cbase(g), _CHUNK)], zsem)
    return 0
  lax.fori_loop(gv, nchunk, zfill, 0)

  def mrow(j, _):
    tvec = h * half + j * _L + iota
    mask_v[pl.ds(j * _L, _L)] = (tvec < total).astype(jnp.int32)
    return 0
  lax.fori_loop(0, half // _L, mrow, 0)
  pltpu.sync_copy(mask_v, mask_hbm.at[pl.ds(b * T + h * half, half)])

  base = b * N
  sgroups = jnp.minimum((vc + 1) * (_CHUNK // _L), T // _L)

  def scan_frames(j, carry):
    a = A_v[pl.ds(j * _L, _L)]
    idxv = jnp.maximum(plsc.cummax(a), carry)
    fidx_v[pl.ds(j * _L, _L)] = base + idxv
    return jnp.max(idxv)
  lax.fori_loop(0, sgroups, scan_frames, jnp.int32(0))

  def gstart(g, p):
    pltpu.async_copy(x_hbm.at[fidx_v.at[pl.ds(cbase(g), _CHUNK)]],
                     gbuf.at[p], gsem.at[p])

  def gwait(g, p):
    pltpu.make_async_copy(x_hbm.at[fidx_v.at[pl.ds(cbase(g), _CHUNK)]],
                          gbuf.at[p], gsem.at[p]).wait()

  def ostart(g, p):
    pltpu.async_copy(gbuf.at[p], out_hbm.at[pl.ds(b * T + cbase(g), _CHUNK)],
                     osem.at[p])

  def owait(g, p):
    pltpu.make_async_copy(gbuf.at[p],
                          out_hbm.at[pl.ds(b * T + cbase(g), _CHUNK)],
                          osem.at[p]).wait()

  @pl.when(gv > 0)
  def _():
    gstart(0, 0)

  def pipe(g, _):
    p = g % 2
    q = 1 - p

    @pl.when(g + 1 < gv)
    def _():
      @pl.when(g >= 1)
      def _():
        owait(g - 1, q)
      gstart(g + 1, q)

    gwait(g, p)
    nvalid = jnp.clip(total - cbase(g), 0, _CHUNK)

    def zrow(r, _):
      for v in range(D // _L):
        gbuf[p, r, pl.ds(v * _L, _L)] = zeros_f
      return 0
    lax.fori_loop(nvalid, _CHUNK, zrow, 0)
    ostart(g, p)
    return 0
  lax.fori_loop(0, gv, pipe, 0)

  @pl.when(gv >= 2)
  def _():
    owait(gv - 2, gv % 2)

  @pl.when(gv >= 1)
  def _():
    owait(gv - 1, (gv + 1) % 2)

  def zdrain(i, _):
    pltpu.make_async_copy(zbuf, out_hbm.at[pl.ds(b * T + cbase(gv + i), _CHUNK)],
                          zsem).wait()
    return 0
  lax.fori_loop(0, nchunk - gv, zdrain, 0)


def kernel(x, durations, target_len):
  B, N, D = x.shape
  T = _T_OUT
  tl_arr = jnp.full((_L,), target_len, jnp.int32)

  mesh = plsc.VectorSubcoreMesh(core_axis_name="c", subcore_axis_name="s")
  out_flat, mask_flat = pl.kernel(
      functools.partial(_lr_body, B, N, D, T),
      out_type=(jax.ShapeDtypeStruct((B * T, D), jnp.float32),
                jax.ShapeDtypeStruct((B * T,), jnp.int32)),
      mesh=mesh,
      compiler_params=pltpu.CompilerParams(needs_layout_passes=False),
      scratch_types=[
          pltpu.VMEM((N,), jnp.int32),
          pltpu.VMEM((_L,), jnp.int32),
          pltpu.VMEM((T,), jnp.int32),
          pltpu.VMEM((T,), jnp.int32),
          pltpu.VMEM((T // 2,), jnp.int32),
          pltpu.VMEM((2, _CHUNK, D), jnp.float32),
          pltpu.VMEM((_CHUNK, D), jnp.float32),
          pltpu.SemaphoreType.DMA((2,)),
          pltpu.SemaphoreType.DMA((2,)),
          pltpu.SemaphoreType.DMA,
      ],
  )(x.reshape(B * N, D), durations.astype(jnp.int32), tl_arr)
  return out_flat.reshape(B, T, D), (mask_flat.reshape(B, T) != 0)
